# Initial kernel scaffold; baseline (speedup 1.0000x reference)
import jax
import jax.numpy as jnp
from jax.experimental import pallas as pl


def kernel(x, V):
    raise NotImplementedError("write your pallas kernel here")


# speedup vs baseline: 1.8220x; 1.8220x over previous
import jax
import jax.numpy as jnp
from jax import lax
from jax.experimental import pallas as pl
from jax.experimental.pallas import tpu as pltpu
from jax.experimental.pallas import tpu_sc as plsc

_HU = (-1000.0, -75.0, 0.0, 15.0, 25.0, 40.0, 50.0, 200.0, 1000.0)


def _dyn_gather(table, idx):
    return lax.gather(
        table,
        idx[:, None],
        dimension_numbers=lax.GatherDimensionNumbers(
            offset_dims=(), collapsed_slice_dims=(0,), start_index_map=(0,)),
        slice_sizes=(1,),
        mode=lax.GatherScatterMode.PROMISE_IN_BOUNDS,
    )


def _body(x_hbm, vt_hbm, out_hbm, x_v, vt_v, out_v):
    b = lax.axis_index("c")
    dp = lax.axis_index("s")
    row0 = b * 64 + dp * 4
    for pd in range(4):
        pltpu.sync_copy(x_hbm.at[row0 + pd], x_v.at[pl.ds(pd * 16384, 16384)])
    pltpu.sync_copy(vt_hbm, vt_v)
    vcols = [vt_v[vd, :] for vd in range(8)]
    iota4 = lax.iota(jnp.int32, 16) * 4

    def combo_body(combo, carry):
        pd = combo >> 4
        ph = (combo >> 2) & 3
        pw = combo & 3
        base = pd * 16384 + ph * 128 + pw

        def chunk_body(c, carry2):
            off = base + (c >> 1) * 512 + (c & 1) * 64
            xv = plsc.load_gather(x_v, [iota4 + off])
            bins = (xv >= _HU[0]).astype(jnp.int32)
            for t in _HU[1:]:
                bins = bins + (xv >= t).astype(jnp.int32)
            for vd in range(8):
                out_v[vd, pl.ds(c * 16, 16)] = _dyn_gather(vcols[vd], bins)
            return carry2

        lax.fori_loop(0, 64, chunk_body, 0)
        pltpu.sync_copy(out_v, out_hbm.at[b, combo, :, pl.ds(dp * 1024, 1024)])
        return carry

    lax.fori_loop(0, 64, combo_body, 0)


def kernel(x, V):
    B, C, D, H, W = x.shape
    x2 = x.reshape(B * D, H * W)
    vt = jnp.zeros((8, 16), jnp.float32).at[:, :10].set(V.T)
    mesh = plsc.VectorSubcoreMesh(core_axis_name="c", subcore_axis_name="s")
    run = pl.kernel(
        _body,
        out_type=jax.ShapeDtypeStruct((B, 64, 8, (D // 4) * 1024), jnp.float32),
        mesh=mesh,
        compiler_params=pltpu.CompilerParams(needs_layout_passes=False),
        scratch_types=[
            pltpu.VMEM((4 * 16384,), jnp.float32),
            pltpu.VMEM((8, 16), jnp.float32),
            pltpu.VMEM((8, 1024), jnp.float32),
        ],
    )
    out = run(x2, vt)
    return out.reshape(B, 512, D // 4, H // 4, W // 4)


# speedup vs baseline: 1.8984x; 1.0419x over previous
import jax
import jax.numpy as jnp
from jax import lax
from jax.experimental import pallas as pl
from jax.experimental.pallas import tpu as pltpu
from jax.experimental.pallas import tpu_sc as plsc

_HU = (-1000.0, -75.0, 0.0, 15.0, 25.0, 40.0, 50.0, 200.0, 1000.0)


def _dyn_gather(table, idx):
    return lax.gather(
        table,
        idx[:, None],
        dimension_numbers=lax.GatherDimensionNumbers(
            offset_dims=(), collapsed_slice_dims=(0,), start_index_map=(0,)),
        slice_sizes=(1,),
        mode=lax.GatherScatterMode.PROMISE_IN_BOUNDS,
    )


def _bins16(xv):
    terms = [(xv >= t).astype(jnp.int32) for t in _HU]
    while len(terms) > 1:
        nxt = [terms[i] + terms[i + 1] for i in range(0, len(terms) - 1, 2)]
        if len(terms) % 2:
            nxt.append(terms[-1])
        terms = nxt
    return terms[0]


_NBUF = 4


def _body(x_hbm, vt_hbm, out_hbm, x_v, vt_v, out_v, sems):
    b = lax.axis_index("c")
    dp = lax.axis_index("s")
    row0 = b * 64 + dp * 4
    for pd in range(4):
        pltpu.sync_copy(x_hbm.at[row0 + pd], x_v.at[pl.ds(pd * 16384, 16384)])
    pltpu.sync_copy(vt_hbm, vt_v)
    vcols = [vt_v[vd, :] for vd in range(8)]
    iota4 = lax.iota(jnp.int32, 16) * 4

    def combo_body(combo, carry):
        pd = combo >> 4
        ph = (combo >> 2) & 3
        pw = combo & 3
        base = pd * 16384 + ph * 128 + pw
        buf = combo & (_NBUF - 1)

        @pl.when(combo >= _NBUF)
        def _():
            pltpu.make_async_copy(
                out_v.at[buf], out_hbm.at[b, combo, :, pl.ds(dp * 1024, 1024)],
                sems.at[buf]).wait()

        def chunk_body(c, carry2):
            off = base + (c >> 1) * 512 + (c & 1) * 64
            xv = plsc.load_gather(x_v, [iota4 + off])
            bins = _bins16(xv)
            for vd in range(8):
                out_v[buf, vd, pl.ds(c * 16, 16)] = _dyn_gather(vcols[vd], bins)
            return carry2

        lax.fori_loop(0, 64, chunk_body, 0, unroll=4)
        pltpu.make_async_copy(
            out_v.at[buf], out_hbm.at[b, combo, :, pl.ds(dp * 1024, 1024)],
            sems.at[buf]).start()
        return carry

    lax.fori_loop(0, 64, combo_body, 0)
    for k in range(_NBUF):
        combo = 64 - _NBUF + k
        buf = combo & (_NBUF - 1)
        pltpu.make_async_copy(
            out_v.at[buf], out_hbm.at[b, combo, :, pl.ds(dp * 1024, 1024)],
            sems.at[buf]).wait()


def kernel(x, V):
    B, C, D, H, W = x.shape
    x2 = x.reshape(B * D, H * W)
    vt = jnp.zeros((8, 16), jnp.float32).at[:, :10].set(V.T)
    mesh = plsc.VectorSubcoreMesh(core_axis_name="c", subcore_axis_name="s")
    run = pl.kernel(
        _body,
        out_type=jax.ShapeDtypeStruct((B, 64, 8, (D // 4) * 1024), jnp.float32),
        mesh=mesh,
        compiler_params=pltpu.CompilerParams(needs_layout_passes=False),
        scratch_types=[
            pltpu.VMEM((4 * 16384,), jnp.float32),
            pltpu.VMEM((8, 16), jnp.float32),
            pltpu.VMEM((_NBUF, 8, 1024), jnp.float32),
            pltpu.SemaphoreType.DMA((_NBUF,)),
        ],
    )
    out = run(x2, vt)
    return out.reshape(B, 512, D // 4, H // 4, W // 4)


# speedup vs baseline: 4.0619x; 2.1396x over previous
import jax
import jax.numpy as jnp
from jax import lax
from jax.experimental import pallas as pl
from jax.experimental.pallas import tpu as pltpu
from jax.experimental.pallas import tpu_sc as plsc

_HU = (-1000.0, -75.0, 0.0, 15.0, 25.0, 40.0, 50.0, 200.0, 1000.0)


def _dyn_gather(table, idx):
    return lax.gather(
        table,
        idx[:, None],
        dimension_numbers=lax.GatherDimensionNumbers(
            offset_dims=(), collapsed_slice_dims=(0,), start_index_map=(0,)),
        slice_sizes=(1,),
        mode=lax.GatherScatterMode.PROMISE_IN_BOUNDS,
    )


def _bins8x(xv):
    terms = [(xv >= t).astype(jnp.int32) for t in _HU]
    while len(terms) > 1:
        nxt = [terms[i] + terms[i + 1] for i in range(0, len(terms) - 1, 2)]
        if len(terms) % 2:
            nxt.append(terms[-1])
        terms = nxt
    return terms[0] * 8


def _body(x_hbm, vf_hbm, out_hbm, x_v, vf_v, slab_v, sems):
    b = lax.axis_index("c")
    dp = lax.axis_index("s")
    row0 = b * 64 + dp * 4
    for pd in range(4):
        pltpu.sync_copy(x_hbm.at[row0 + pd], x_v.at[pl.ds(pd * 16384, 16384)])
    pltpu.sync_copy(vf_hbm, vf_v)

    lanes = lax.iota(jnp.int32, 16)
    vd_iota = lanes & 7
    halfsel = (lanes >> 3) & 1
    maps = [[(4 * rr + 2 * k) + halfsel for k in range(2)] for rr in range(4)]

    def slab_body(hp, carry):
        buf = hp & 1

        @pl.when(hp >= 2)
        def _():
            pltpu.make_async_copy(
                slab_v.at[buf], out_hbm.at[b, dp, hp], sems.at[buf]).wait()

        def vreg_body(v, carry2):
            pd = v >> 5
            ph = (v >> 3) & 3
            j = v & 7
            xv = x_v[pl.ds(pd * 16384 + (4 * hp + ph) * 128 + 16 * j, 16)]
            bins8 = _bins8x(xv)
            base = ((j >> 1) * 4096 + pd * 1024 + (j & 1) * 512 + ph * 32)
            for rr in range(4):
                for k in range(2):
                    idx = _dyn_gather(bins8, maps[rr][k]) + vd_iota
                    val = plsc.load_gather(vf_v, [idx])
                    slab_v[buf, pl.ds(base + rr * 128 + 16 * k, 16)] = val
            return carry2

        lax.fori_loop(0, 128, vreg_body, 0, unroll=2)
        pltpu.make_async_copy(
            slab_v.at[buf], out_hbm.at[b, dp, hp], sems.at[buf]).start()
        return carry

    lax.fori_loop(0, 32, slab_body, 0)
    for hp in (30, 31):
        pltpu.make_async_copy(
            slab_v.at[hp & 1], out_hbm.at[b, dp, hp], sems.at[hp & 1]).wait()


def kernel(x, V):
    B, C, D, H, W = x.shape
    x2 = x.reshape(B * D, H * W)
    vf = V.reshape(80)
    mesh = plsc.VectorSubcoreMesh(core_axis_name="c", subcore_axis_name="s")
    run = pl.kernel(
        _body,
        out_type=jax.ShapeDtypeStruct((B, 16, 32, 16384), jnp.float32),
        mesh=mesh,
        compiler_params=pltpu.CompilerParams(needs_layout_passes=False),
        scratch_types=[
            pltpu.VMEM((4 * 16384,), jnp.float32),
            pltpu.VMEM((80,), jnp.float32),
            pltpu.VMEM((2, 16384), jnp.float32),
            pltpu.SemaphoreType.DMA((2,)),
        ],
    )
    out = run(x2, vf)
    out = out.reshape(B, 16, 32, 4, 4, 8, 128)
    out = out.transpose(0, 4, 6, 1, 2, 3, 5)
    return out.reshape(B, 512, D // 4, H // 4, W // 4)
